# baseline (speedup 1.0000x reference)
import functools

import jax
import jax.numpy as jnp
from jax import lax
from jax.experimental import pallas as pl
from jax.experimental.pallas import tpu as pltpu
from jax.experimental.pallas import tpu_sc as plsc


_LANES = 128
_CHUNK = 160


def kernel(x, table):
    B, S = x.shape
    V, E = table.shape
    idx = x.reshape(B, 1, S)
    tab_p = jnp.pad(table, ((0, 0), (0, _LANES - E)))
    w0 = _LANES
    w1 = S - _LANES

    mesh = plsc.VectorSubcoreMesh(core_axis_name="core",
                                  subcore_axis_name="subcore")

    @functools.partial(
        pl.kernel,
        out_type=jax.ShapeDtypeStruct((B, S, E), table.dtype),
        mesh=mesh,
        scratch_types=[
            pltpu.VMEM_SHARED((V, E), jnp.float32),
            pltpu.VMEM((_CHUNK, _LANES), jnp.float32),
            pltpu.VMEM((_CHUNK, E), jnp.float32),
        ],
    )
    def gather_kernel(tab_hbm, i_hbm, o_hbm, tab_shared, tstage, tpack):
        sid = lax.axis_index("subcore")

        @pl.when(sid == 0)
        def _stage_table():
            @pl.loop(0, V, step=_CHUNK)
            def _chunk(k):
                pltpu.sync_copy(tab_hbm.at[pl.ds(k, _CHUNK)], tstage)

                @pl.loop(0, _CHUNK)
                def _row(r):
                    for c in range(E // 16):
                        tpack.at[r, pl.ds(c * 16, 16)][...] = (
                            tstage.at[r, pl.ds(c * 16, 16)][...])

                pltpu.sync_copy(tpack, tab_shared.at[pl.ds(k, _CHUNK)])

        plsc.subcore_barrier()

        def body(i_vmem, o_vmem):
            pltpu.sync_copy(tab_shared.at[i_vmem.at[0, 0, pl.ds(0, w0)]],
                            o_vmem.at[0, pl.ds(0, w0)])
            pltpu.sync_copy(tab_shared.at[i_vmem.at[0, 0, pl.ds(w0, w1)]],
                            o_vmem.at[0, pl.ds(w0, w1)])

        pltpu.emit_pipeline(
            body,
            grid=(B,),
            in_specs=[pl.BlockSpec((1, 1, S), index_map=lambda i: (i, 0, 0))],
            out_specs=[pl.BlockSpec((1, S, E), index_map=lambda i: (i, 0, 0))],
            core_axis_name=("core", "subcore"),
            dimension_semantics=(pltpu.PARALLEL,),
        )(i_hbm, o_hbm)

    return gather_kernel(tab_p, idx)


# speedup vs baseline: 1.2432x; 1.2432x over previous
import functools

import jax
import jax.numpy as jnp
from jax import lax
from jax.experimental import pallas as pl
from jax.experimental.pallas import tpu as pltpu
from jax.experimental.pallas import tpu_sc as plsc


_LANES = 128
_ROWS = 2


def kernel(x, table):
    B, S = x.shape
    V, E = table.shape
    idx = x.reshape(B // _ROWS, _ROWS, S)
    tab_p = jnp.pad(table, ((0, 0), (0, _LANES - E)))
    w0 = _LANES
    w1 = S - _LANES

    mesh = plsc.VectorSubcoreMesh(core_axis_name="core",
                                  subcore_axis_name="subcore")

    @functools.partial(
        pl.kernel,
        out_type=jax.ShapeDtypeStruct((B, S, _LANES), table.dtype),
        mesh=mesh,
        scratch_types=[
            pltpu.VMEM_SHARED((V, _LANES), jnp.float32),
            pltpu.SemaphoreType.DMA,
        ],
    )
    def gather_kernel(tab_hbm, i_hbm, o_hbm, tab_shared, sem):
        sid = lax.axis_index("subcore")

        @pl.when(sid == 0)
        def _stage_table():
            pltpu.sync_copy(tab_hbm, tab_shared)

        plsc.subcore_barrier()

        def body(i_vmem, o_vmem):
            copies = []
            for r in range(_ROWS):
                copies.append(pltpu.async_copy(
                    tab_shared.at[i_vmem.at[0, r, pl.ds(0, w0)]],
                    o_vmem.at[r, pl.ds(0, w0)], sem))
                copies.append(pltpu.async_copy(
                    tab_shared.at[i_vmem.at[0, r, pl.ds(w0, w1)]],
                    o_vmem.at[r, pl.ds(w0, w1)], sem))
            for c in copies:
                c.wait()

        pltpu.emit_pipeline(
            body,
            grid=(B // _ROWS,),
            in_specs=[pl.BlockSpec((1, _ROWS, S),
                                   index_map=lambda i: (i, 0, 0))],
            out_specs=[pl.BlockSpec((_ROWS, S, _LANES),
                                    index_map=lambda i: (i, 0, 0))],
            core_axis_name=("core", "subcore"),
            dimension_semantics=(pltpu.PARALLEL,),
        )(i_hbm, o_hbm)

    return gather_kernel(tab_p, idx)[:, :, :E]
